# Initial kernel scaffold; baseline (speedup 1.0000x reference)
import jax
import jax.numpy as jnp
from jax.experimental import pallas as pl


def kernel(x, pe_weight):
    raise NotImplementedError("write your pallas kernel here")


# speedup vs baseline: 1.4709x; 1.4709x over previous
import jax
import jax.numpy as jnp
from jax.experimental import pallas as pl


def _add_body(x_ref, pe_ref, o_ref):
    o_ref[...] = x_ref[...] + pe_ref[...]


def kernel(x, pe_weight):
    B, S, D = x.shape
    bs = 256
    pe = pe_weight[:S]
    return pl.pallas_call(
        _add_body,
        grid=(S // bs, B),
        in_specs=[
            pl.BlockSpec((1, bs, D), lambda s, b: (b, s, 0)),
            pl.BlockSpec((bs, D), lambda s, b: (s, 0)),
        ],
        out_specs=pl.BlockSpec((1, bs, D), lambda s, b: (b, s, 0)),
        out_shape=jax.ShapeDtypeStruct((B, S, D), x.dtype),
    )(x, pe)
